# baseline (speedup 1.0000x reference)
import functools

import jax
import jax.numpy as jnp
from jax.experimental import pallas as pl
from jax.experimental.pallas import tpu as pltpu

B, S, E = 2, 256, 256
TI = 32
NTAP = 8


def _dot(a, b):
    return jax.lax.dot_general(a, b, (((1,), (0,)), ((), ())),
                               preferred_element_type=jnp.float32)


def _layer_norm(x, g, b, eps=1e-5):
    m = jnp.mean(x, axis=-1, keepdims=True)
    xc = x - m
    v = jnp.mean(xc * xc, axis=-1, keepdims=True)
    return xc * jax.lax.rsqrt(v + eps) * g + b


def _fused_kernel(adj_ref, x_ref, ee_W1_ref, ee_b1_ref, ee_W2_ref, ee_b2_ref,
                  gin0_W1_ref, gin0_b1_ref, gin0_W2_ref, gin0_b2_ref,
                  gin1_W1_ref, gin1_b1_ref, gin1_W2_ref, gin1_b2_ref,
                  ln0_g_ref, ln0_b_ref, ln1_g_ref, ln1_b_ref, res1_ref,
                  m_ref, beff_ref, sf_W2_ref, sf_b2_ref,
                  final_ref, msum_ref, e_ref):
    x = x_ref[0]
    adj = adj_ref[0]

    a_rows = _dot(x, ee_W1_ref[:E, :]) + ee_b1_ref[...]
    b_rows = _dot(x, ee_W1_ref[E:, :])
    w2 = ee_W2_ref[...]
    b2 = ee_b2_ref[...]
    for i0 in range(0, S, TI):
        h = jax.nn.relu(a_rows[i0:i0 + TI, None, :] + b_rows[None, :, :])
        h = h.reshape(TI * S, E)
        t = jnp.tanh(_dot(h, w2) + b2)
        e_ref[i0:i0 + TI, :] = jnp.mean(t, axis=-1).reshape(TI, S)
    e = e_ref[...]
    adj_e = adj * (1.0 + e)

    msum_ref[0] = (jnp.sum(adj_e, axis=1, keepdims=True)
                   + jnp.sum(adj_e, axis=0).reshape(S, 1))

    zpad = jnp.zeros((4, E), jnp.float32)
    xp = jnp.concatenate([zpad, x, zpad], axis=0)
    pr = beff_ref[...]
    for j in range(NTAP):
        pr = pr + _dot(xp[j:j + S, :], m_ref[j])
    ms = _dot(jax.nn.relu(pr), sf_W2_ref[...]) + sf_b2_ref[...]

    gin_in0 = x + _dot(adj_e, x)
    lo = _dot(jax.nn.relu(_dot(gin_in0, gin0_W1_ref[...]) + gin0_b1_ref[...]),
              gin0_W2_ref[...]) + gin0_b2_ref[...]
    r0 = jax.nn.relu(_layer_norm(lo, ln0_g_ref[...], ln0_b_ref[...]))

    gin_in1 = r0 + _dot(adj_e, r0)
    lo = _dot(jax.nn.relu(_dot(gin_in1, gin1_W1_ref[...]) + gin1_b1_ref[...]),
              gin1_W2_ref[...]) + gin1_b2_ref[...]
    rw = jax.nn.sigmoid(res1_ref[0, 0])
    lo = rw * lo + (1.0 - rw) * r0
    out = jax.nn.relu(_layer_norm(lo, ln1_g_ref[...], ln1_b_ref[...]))

    final_ref[0] = jnp.concatenate([out, ms], axis=-1)


@functools.partial(jax.jit, static_argnames=("interpret",))
def _run(adj, inputs, ee_W1, ee_b1, ee_W2, ee_b2, gin0_W1, gin0_b1, gin0_W2,
         gin0_b2, gin1_W1, gin1_b1, gin1_W2, gin1_b2, ln0_g, ln0_b, ln1_g,
         ln1_b, res1, m_taps, b_eff, sf_W2, sf_b2, interpret=False):
    def full(shape):
        return pl.BlockSpec(shape, lambda b: (0,) * len(shape))

    in_specs = [
        pl.BlockSpec((1, S, S), lambda b: (b, 0, 0)),
        pl.BlockSpec((1, S, E), lambda b: (b, 0, 0)),
        full((2 * E, E)), full((1, E)), full((E, E)), full((1, E)),
        full((E, E)), full((1, E)), full((E, E)), full((1, E)),
        full((E, E)), full((1, E)), full((E, E)), full((1, E)),
        full((1, E)), full((1, E)), full((1, E)), full((1, E)),
        full((1, 1)),
        full((NTAP, E, E)), full((1, E)), full((E, E)), full((1, E)),
    ]
    out_specs = [
        pl.BlockSpec((1, S, 2 * E), lambda b: (b, 0, 0)),
        pl.BlockSpec((1, S, 1), lambda b: (b, 0, 0)),
    ]
    final, msum = pl.pallas_call(
        _fused_kernel,
        grid=(B,),
        in_specs=in_specs,
        out_specs=out_specs,
        out_shape=[
            jax.ShapeDtypeStruct((B, S, 2 * E), jnp.float32),
            jax.ShapeDtypeStruct((B, S, 1), jnp.float32),
        ],
        scratch_shapes=[pltpu.VMEM((S, S), jnp.float32)],
        compiler_params=pltpu.CompilerParams(
            dimension_semantics=("parallel",)),
        interpret=interpret,
    )(adj, inputs, ee_W1, ee_b1, ee_W2, ee_b2, gin0_W1, gin0_b1, gin0_W2,
      gin0_b2, gin1_W1, gin1_b1, gin1_W2, gin1_b2, ln0_g, ln0_b, ln1_g,
      ln1_b, res1, m_taps, b_eff, sf_W2, sf_b2)
    return final, msum


def kernel(adj, inputs, ee_W1, ee_b1, ee_W2, ee_b2, gin0_W1, gin0_b1,
           gin0_W2, gin0_b2, gin1_W1, gin1_b1, gin1_W2, gin1_b2, ln0_g,
           ln0_b, ln1_g, ln1_b, res0, res1, conv1_W, conv1_b, conv2_W,
           conv2_b, conv3_W, conv3_b, sf_W1, sf_b1, sf_W2, sf_b2,
           interpret=False):
    row = lambda v: v.reshape(1, E)
    m_taps = jnp.zeros((NTAP, E, E), jnp.float32)
    b_eff = sf_b1
    for wk, bk, pk, off in ((conv1_W, conv1_b, 1, 0),
                            (conv2_W, conv2_b, 2, E),
                            (conv3_W, conv3_b, 4, 2 * E)):
        k = wk.shape[2]
        sf = sf_W1[off:off + E]
        m_taps = m_taps.at[4 - pk:4 - pk + k].add(
            jnp.einsum('oit,oe->tie', wk, sf, precision='highest'))
        b_eff = b_eff + bk @ sf
    final, msum = _run(
        adj, inputs, ee_W1, row(ee_b1), ee_W2, row(ee_b2),
        gin0_W1, row(gin0_b1), gin0_W2, row(gin0_b2),
        gin1_W1, row(gin1_b1), gin1_W2, row(gin1_b2),
        row(ln0_g), row(ln0_b), row(ln1_g), row(ln1_b),
        res1.reshape(1, 1), m_taps, row(b_eff),
        sf_W2, row(sf_b2), interpret=interpret)
    mask = msum == 0.0
    return final, mask


# speedup vs baseline: 1.0619x; 1.0619x over previous
import functools

import jax
import jax.numpy as jnp
from jax.experimental import pallas as pl
from jax.experimental.pallas import tpu as pltpu

B, S, E = 2, 256, 256
TI = 32
NTAP = 8


def _dot(a, b):
    return jax.lax.dot_general(a, b, (((1,), (0,)), ((), ())),
                               preferred_element_type=jnp.float32)


def _layer_norm(x, g, b, eps=1e-5):
    m = jnp.mean(x, axis=-1, keepdims=True)
    xc = x - m
    v = jnp.mean(xc * xc, axis=-1, keepdims=True)
    return xc * jax.lax.rsqrt(v + eps) * g + b


def _fused_kernel(adj_ref, x_ref, ee_W1_ref, ee_b1_ref, ee_W2_ref, ee_b2_ref,
                  gin0_W1_ref, gin0_b1_ref, gin0_W2_ref, gin0_b2_ref,
                  gin1_W1_ref, gin1_b1_ref, gin1_W2_ref, gin1_b2_ref,
                  ln0_g_ref, ln0_b_ref, ln1_g_ref, ln1_b_ref, res1_ref,
                  m_ref, beff_ref, sf_W2_ref, sf_b2_ref,
                  final_ref, msum_ref, e_ref):
    x = x_ref[0]
    adj = adj_ref[0]

    a_rows = _dot(x, ee_W1_ref[:E, :]) + ee_b1_ref[...]
    b_rows = _dot(x, ee_W1_ref[E:, :])
    w2 = ee_W2_ref[...]
    b2 = ee_b2_ref[...]
    for i0 in range(0, S, TI):
        h = jax.nn.relu(a_rows[i0:i0 + TI, None, :] + b_rows[None, :, :])
        h = h.reshape(TI * S, E)
        t = jnp.tanh(_dot(h, w2) + b2)
        e_ref[i0:i0 + TI, :] = jnp.mean(t, axis=-1).reshape(TI, S)
    e = e_ref[...]
    adj_e = adj * (1.0 + e)

    msum_ref[0] = (jnp.sum(adj_e, axis=1, keepdims=True)
                   + jnp.sum(adj_e, axis=0).reshape(S, 1))

    zpad = jnp.zeros((4, E), jnp.float32)
    xp = jnp.concatenate([zpad, x, zpad], axis=0)
    pr = beff_ref[...]
    for j in range(NTAP):
        pr = pr + _dot(xp[j:j + S, :], m_ref[j])
    ms = _dot(jax.nn.relu(pr), sf_W2_ref[...]) + sf_b2_ref[...]

    gin_in0 = x + _dot(adj_e, x)
    lo = _dot(jax.nn.relu(_dot(gin_in0, gin0_W1_ref[...]) + gin0_b1_ref[...]),
              gin0_W2_ref[...]) + gin0_b2_ref[...]
    r0 = jax.nn.relu(_layer_norm(lo, ln0_g_ref[...], ln0_b_ref[...]))

    gin_in1 = r0 + _dot(adj_e, r0)
    lo = _dot(jax.nn.relu(_dot(gin_in1, gin1_W1_ref[...]) + gin1_b1_ref[...]),
              gin1_W2_ref[...]) + gin1_b2_ref[...]
    rw = jax.nn.sigmoid(res1_ref[0, 0])
    lo = rw * lo + (1.0 - rw) * r0
    out = jax.nn.relu(_layer_norm(lo, ln1_g_ref[...], ln1_b_ref[...]))

    final_ref[0] = jnp.concatenate([out, ms], axis=-1)


@functools.partial(jax.jit, static_argnames=("interpret",))
def _run(adj, inputs, ee_W1, ee_b1, ee_W2, ee_b2, gin0_W1, gin0_b1, gin0_W2,
         gin0_b2, gin1_W1, gin1_b1, gin1_W2, gin1_b2, ln0_g, ln0_b, ln1_g,
         ln1_b, res1, m_taps, b_eff, sf_W2, sf_b2, interpret=False):
    def full(shape):
        return pl.BlockSpec(shape, lambda b: (0,) * len(shape))

    in_specs = [
        pl.BlockSpec((1, S, S), lambda b: (b, 0, 0)),
        pl.BlockSpec((1, S, E), lambda b: (b, 0, 0)),
        full((2 * E, E)), full((1, E)), full((E, E)), full((1, E)),
        full((E, E)), full((1, E)), full((E, E)), full((1, E)),
        full((E, E)), full((1, E)), full((E, E)), full((1, E)),
        full((1, E)), full((1, E)), full((1, E)), full((1, E)),
        full((1, 1)),
        full((NTAP, E, E)), full((1, E)), full((E, E)), full((1, E)),
    ]
    out_specs = [
        pl.BlockSpec((1, S, 2 * E), lambda b: (b, 0, 0)),
        pl.BlockSpec((1, S, 1), lambda b: (b, 0, 0)),
    ]
    final, msum = pl.pallas_call(
        _fused_kernel,
        grid=(B,),
        in_specs=in_specs,
        out_specs=out_specs,
        out_shape=[
            jax.ShapeDtypeStruct((B, S, 2 * E), jnp.float32),
            jax.ShapeDtypeStruct((B, S, 1), jnp.float32),
        ],
        scratch_shapes=[pltpu.VMEM((S, S), jnp.float32)],
        compiler_params=pltpu.CompilerParams(
            dimension_semantics=("parallel",)),
        interpret=interpret,
    )(adj, inputs, ee_W1, ee_b1, ee_W2, ee_b2, gin0_W1, gin0_b1, gin0_W2,
      gin0_b2, gin1_W1, gin1_b1, gin1_W2, gin1_b2, ln0_g, ln0_b, ln1_g,
      ln1_b, res1, m_taps, b_eff, sf_W2, sf_b2)
    return final, msum


def kernel(adj, inputs, ee_W1, ee_b1, ee_W2, ee_b2, gin0_W1, gin0_b1,
           gin0_W2, gin0_b2, gin1_W1, gin1_b1, gin1_W2, gin1_b2, ln0_g,
           ln0_b, ln1_g, ln1_b, res0, res1, conv1_W, conv1_b, conv2_W,
           conv2_b, conv3_W, conv3_b, sf_W1, sf_b1, sf_W2, sf_b2,
           interpret=False):
    row = lambda v: v.reshape(1, E)
    m_taps = jnp.zeros((NTAP, E, E), jnp.float32)
    b_eff = sf_b1
    for wk, bk, pk, off in ((conv1_W, conv1_b, 1, 0),
                            (conv2_W, conv2_b, 2, E),
                            (conv3_W, conv3_b, 4, 2 * E)):
        k = wk.shape[2]
        sf = sf_W1[off:off + E]
        m_taps = m_taps.at[4 - pk:4 - pk + k].add(
            jnp.einsum('oit,oe->tie', wk, sf, precision='high'))
        b_eff = b_eff + bk @ sf
    final, msum = _run(
        adj, inputs, ee_W1, row(ee_b1), ee_W2, row(ee_b2),
        gin0_W1, row(gin0_b1), gin0_W2, row(gin0_b2),
        gin1_W1, row(gin1_b1), gin1_W2, row(gin1_b2),
        row(ln0_g), row(ln0_b), row(ln1_g), row(ln1_b),
        res1.reshape(1, 1), m_taps, row(b_eff),
        sf_W2, row(sf_b2), interpret=interpret)
    mask = msum == 0.0
    return final, mask


# speedup vs baseline: 1.1446x; 1.0779x over previous
import functools

import jax
import jax.numpy as jnp
from jax.experimental import pallas as pl
from jax.experimental.pallas import tpu as pltpu

B, S, E = 2, 256, 256
TI = 32
NTAP = 8


def _dot(a, b):
    return jax.lax.dot_general(a, b, (((1,), (0,)), ((), ())),
                               preferred_element_type=jnp.float32)


def _layer_norm(x, g, b, eps=1e-5):
    m = jnp.mean(x, axis=-1, keepdims=True)
    xc = x - m
    v = jnp.mean(xc * xc, axis=-1, keepdims=True)
    return xc * jax.lax.rsqrt(v + eps) * g + b


def _fused_kernel(adj_ref, x_ref, ee_W1_ref, ee_b1_ref, ee_W2_ref, ee_b2_ref,
                  gin0_W1_ref, gin0_b1_ref, gin0_W2_ref, gin0_b2_ref,
                  gin1_W1_ref, gin1_b1_ref, gin1_W2_ref, gin1_b2_ref,
                  ln0_g_ref, ln0_b_ref, ln1_g_ref, ln1_b_ref, res1_ref,
                  m_ref, beff_ref, sf_W2_ref, sf_b2_ref,
                  final_ref, msum_ref, e_ref):
    x = x_ref[0]
    adj = adj_ref[0]

    a_rows = _dot(x, ee_W1_ref[:E, :]) + ee_b1_ref[...]
    b_rows = _dot(x, ee_W1_ref[E:, :])
    w2 = ee_W2_ref[...]
    b2 = ee_b2_ref[...]
    for i0 in range(0, S, TI):
        h = jax.nn.relu(a_rows[i0:i0 + TI, None, :] + b_rows[None, :, :])
        h = h.reshape(TI * S, E)
        t = jnp.tanh(_dot(h, w2) + b2)
        e_ref[i0:i0 + TI, :] = jnp.mean(t, axis=-1).reshape(TI, S)
    e = e_ref[...]
    adj_e = adj * (1.0 + e)

    msum_ref[0] = (jnp.sum(adj_e, axis=1, keepdims=True)
                   + jnp.sum(adj_e, axis=0).reshape(S, 1)) == 0.0

    zpad = jnp.zeros((4, E), jnp.float32)
    xp = jnp.concatenate([zpad, x, zpad], axis=0)
    pr = beff_ref[...]
    for j in range(NTAP):
        pr = pr + _dot(xp[j:j + S, :], m_ref[j])
    ms = _dot(jax.nn.relu(pr), sf_W2_ref[...]) + sf_b2_ref[...]

    gin_in0 = x + _dot(adj_e, x)
    lo = _dot(jax.nn.relu(_dot(gin_in0, gin0_W1_ref[...]) + gin0_b1_ref[...]),
              gin0_W2_ref[...]) + gin0_b2_ref[...]
    r0 = jax.nn.relu(_layer_norm(lo, ln0_g_ref[...], ln0_b_ref[...]))

    gin_in1 = r0 + _dot(adj_e, r0)
    lo = _dot(jax.nn.relu(_dot(gin_in1, gin1_W1_ref[...]) + gin1_b1_ref[...]),
              gin1_W2_ref[...]) + gin1_b2_ref[...]
    rw = jax.nn.sigmoid(res1_ref[0, 0])
    lo = rw * lo + (1.0 - rw) * r0
    out = jax.nn.relu(_layer_norm(lo, ln1_g_ref[...], ln1_b_ref[...]))

    final_ref[0] = jnp.concatenate([out, ms], axis=-1)


@functools.partial(jax.jit, static_argnames=("interpret",))
def _run(adj, inputs, ee_W1, ee_b1, ee_W2, ee_b2, gin0_W1, gin0_b1, gin0_W2,
         gin0_b2, gin1_W1, gin1_b1, gin1_W2, gin1_b2, ln0_g, ln0_b, ln1_g,
         ln1_b, res1, m_taps, b_eff, sf_W2, sf_b2, interpret=False):
    def full(shape):
        return pl.BlockSpec(shape, lambda b: (0,) * len(shape))

    in_specs = [
        pl.BlockSpec((1, S, S), lambda b: (b, 0, 0)),
        pl.BlockSpec((1, S, E), lambda b: (b, 0, 0)),
        full((2 * E, E)), full((1, E)), full((E, E)), full((1, E)),
        full((E, E)), full((1, E)), full((E, E)), full((1, E)),
        full((E, E)), full((1, E)), full((E, E)), full((1, E)),
        full((1, E)), full((1, E)), full((1, E)), full((1, E)),
        full((1, 1)),
        full((NTAP, E, E)), full((1, E)), full((E, E)), full((1, E)),
    ]
    out_specs = [
        pl.BlockSpec((1, S, 2 * E), lambda b: (b, 0, 0)),
        pl.BlockSpec((1, S, 1), lambda b: (b, 0, 0)),
    ]
    final, msum = pl.pallas_call(
        _fused_kernel,
        grid=(B,),
        in_specs=in_specs,
        out_specs=out_specs,
        out_shape=[
            jax.ShapeDtypeStruct((B, S, 2 * E), jnp.float32),
            jax.ShapeDtypeStruct((B, S, 1), jnp.bool_),
        ],
        scratch_shapes=[pltpu.VMEM((S, S), jnp.float32)],
        compiler_params=pltpu.CompilerParams(
            dimension_semantics=("parallel",)),
        interpret=interpret,
    )(adj, inputs, ee_W1, ee_b1, ee_W2, ee_b2, gin0_W1, gin0_b1, gin0_W2,
      gin0_b2, gin1_W1, gin1_b1, gin1_W2, gin1_b2, ln0_g, ln0_b, ln1_g,
      ln1_b, res1, m_taps, b_eff, sf_W2, sf_b2)
    return final, msum


def kernel(adj, inputs, ee_W1, ee_b1, ee_W2, ee_b2, gin0_W1, gin0_b1,
           gin0_W2, gin0_b2, gin1_W1, gin1_b1, gin1_W2, gin1_b2, ln0_g,
           ln0_b, ln1_g, ln1_b, res0, res1, conv1_W, conv1_b, conv2_W,
           conv2_b, conv3_W, conv3_b, sf_W1, sf_b1, sf_W2, sf_b2,
           interpret=False):
    row = lambda v: v.reshape(1, E)
    m_taps = jnp.einsum('oit,oe->tie', conv3_W, sf_W1[2 * E:])
    b_eff = sf_b1 + conv3_b @ sf_W1[2 * E:]
    for wk, bk, pk, off in ((conv1_W, conv1_b, 1, 0),
                            (conv2_W, conv2_b, 2, E)):
        k = wk.shape[2]
        sf = sf_W1[off:off + E]
        m_taps = m_taps.at[4 - pk:4 - pk + k].add(
            jnp.einsum('oit,oe->tie', wk, sf))
        b_eff = b_eff + bk @ sf
    final, mask = _run(
        adj, inputs, ee_W1, row(ee_b1), ee_W2, row(ee_b2),
        gin0_W1, row(gin0_b1), gin0_W2, row(gin0_b2),
        gin1_W1, row(gin1_b1), gin1_W2, row(gin1_b2),
        row(ln0_g), row(ln0_b), row(ln1_g), row(ln1_b),
        res1.reshape(1, 1), m_taps, row(b_eff),
        sf_W2, row(sf_b2), interpret=interpret)
    return final, mask


# speedup vs baseline: 1.1468x; 1.0019x over previous
import functools

import jax
import jax.numpy as jnp
from jax.experimental import pallas as pl
from jax.experimental.pallas import tpu as pltpu

B, S, E = 2, 256, 256
TI = 64
NTAP = 8


def _dot(a, b):
    return jax.lax.dot_general(a, b, (((1,), (0,)), ((), ())),
                               preferred_element_type=jnp.float32)


def _layer_norm(x, g, b, eps=1e-5):
    m = jnp.mean(x, axis=-1, keepdims=True)
    xc = x - m
    v = jnp.mean(xc * xc, axis=-1, keepdims=True)
    return xc * jax.lax.rsqrt(v + eps) * g + b


def _fused_kernel(adj_ref, x_ref, ee_W1_ref, ee_b1_ref, ee_W2_ref, ee_b2_ref,
                  gin0_W1_ref, gin0_b1_ref, gin0_W2_ref, gin0_b2_ref,
                  gin1_W1_ref, gin1_b1_ref, gin1_W2_ref, gin1_b2_ref,
                  ln0_g_ref, ln0_b_ref, ln1_g_ref, ln1_b_ref, res1_ref,
                  m_ref, beff_ref, sf_W2_ref, sf_b2_ref,
                  final_ref, msum_ref, e_ref):
    x = x_ref[0]
    adj = adj_ref[0]

    a_rows = _dot(x, ee_W1_ref[:E, :]) + ee_b1_ref[...]
    b_rows = _dot(x, ee_W1_ref[E:, :])
    w2 = ee_W2_ref[...]
    b2 = ee_b2_ref[...]
    for i0 in range(0, S, TI):
        h = jax.nn.relu(a_rows[i0:i0 + TI, None, :] + b_rows[None, :, :])
        h = h.reshape(TI * S, E)
        t = jnp.tanh(_dot(h, w2) + b2)
        e_ref[i0:i0 + TI, :] = jnp.mean(t, axis=-1).reshape(TI, S)
    e = e_ref[...]
    adj_e = adj * (1.0 + e)

    msum_ref[0] = (jnp.sum(adj_e, axis=1, keepdims=True)
                   + jnp.sum(adj_e, axis=0).reshape(S, 1)) == 0.0

    zpad = jnp.zeros((4, E), jnp.float32)
    xp = jnp.concatenate([zpad, x, zpad], axis=0)
    pr = beff_ref[...]
    for j in range(NTAP):
        pr = pr + _dot(xp[j:j + S, :], m_ref[j])
    ms = _dot(jax.nn.relu(pr), sf_W2_ref[...]) + sf_b2_ref[...]

    gin_in0 = x + _dot(adj_e, x)
    lo = _dot(jax.nn.relu(_dot(gin_in0, gin0_W1_ref[...]) + gin0_b1_ref[...]),
              gin0_W2_ref[...]) + gin0_b2_ref[...]
    r0 = jax.nn.relu(_layer_norm(lo, ln0_g_ref[...], ln0_b_ref[...]))

    gin_in1 = r0 + _dot(adj_e, r0)
    lo = _dot(jax.nn.relu(_dot(gin_in1, gin1_W1_ref[...]) + gin1_b1_ref[...]),
              gin1_W2_ref[...]) + gin1_b2_ref[...]
    rw = jax.nn.sigmoid(res1_ref[0, 0])
    lo = rw * lo + (1.0 - rw) * r0
    out = jax.nn.relu(_layer_norm(lo, ln1_g_ref[...], ln1_b_ref[...]))

    final_ref[0] = jnp.concatenate([out, ms], axis=-1)


@functools.partial(jax.jit, static_argnames=("interpret",))
def _run(adj, inputs, ee_W1, ee_b1, ee_W2, ee_b2, gin0_W1, gin0_b1, gin0_W2,
         gin0_b2, gin1_W1, gin1_b1, gin1_W2, gin1_b2, ln0_g, ln0_b, ln1_g,
         ln1_b, res1, m_taps, b_eff, sf_W2, sf_b2, interpret=False):
    def full(shape):
        return pl.BlockSpec(shape, lambda b: (0,) * len(shape))

    in_specs = [
        pl.BlockSpec((1, S, S), lambda b: (b, 0, 0)),
        pl.BlockSpec((1, S, E), lambda b: (b, 0, 0)),
        full((2 * E, E)), full((1, E)), full((E, E)), full((1, E)),
        full((E, E)), full((1, E)), full((E, E)), full((1, E)),
        full((E, E)), full((1, E)), full((E, E)), full((1, E)),
        full((1, E)), full((1, E)), full((1, E)), full((1, E)),
        full((1, 1)),
        full((NTAP, E, E)), full((1, E)), full((E, E)), full((1, E)),
    ]
    out_specs = [
        pl.BlockSpec((1, S, 2 * E), lambda b: (b, 0, 0)),
        pl.BlockSpec((1, S, 1), lambda b: (b, 0, 0)),
    ]
    final, msum = pl.pallas_call(
        _fused_kernel,
        grid=(B,),
        in_specs=in_specs,
        out_specs=out_specs,
        out_shape=[
            jax.ShapeDtypeStruct((B, S, 2 * E), jnp.float32),
            jax.ShapeDtypeStruct((B, S, 1), jnp.bool_),
        ],
        scratch_shapes=[pltpu.VMEM((S, S), jnp.float32)],
        compiler_params=pltpu.CompilerParams(
            dimension_semantics=("parallel",)),
        interpret=interpret,
    )(adj, inputs, ee_W1, ee_b1, ee_W2, ee_b2, gin0_W1, gin0_b1, gin0_W2,
      gin0_b2, gin1_W1, gin1_b1, gin1_W2, gin1_b2, ln0_g, ln0_b, ln1_g,
      ln1_b, res1, m_taps, b_eff, sf_W2, sf_b2)
    return final, msum


def kernel(adj, inputs, ee_W1, ee_b1, ee_W2, ee_b2, gin0_W1, gin0_b1,
           gin0_W2, gin0_b2, gin1_W1, gin1_b1, gin1_W2, gin1_b2, ln0_g,
           ln0_b, ln1_g, ln1_b, res0, res1, conv1_W, conv1_b, conv2_W,
           conv2_b, conv3_W, conv3_b, sf_W1, sf_b1, sf_W2, sf_b2,
           interpret=False):
    row = lambda v: v.reshape(1, E)
    m_taps = jnp.einsum('oit,oe->tie', conv3_W, sf_W1[2 * E:])
    b_eff = sf_b1 + conv3_b @ sf_W1[2 * E:]
    for wk, bk, pk, off in ((conv1_W, conv1_b, 1, 0),
                            (conv2_W, conv2_b, 2, E)):
        k = wk.shape[2]
        sf = sf_W1[off:off + E]
        m_taps = m_taps.at[4 - pk:4 - pk + k].add(
            jnp.einsum('oit,oe->tie', wk, sf))
        b_eff = b_eff + bk @ sf
    final, mask = _run(
        adj, inputs, ee_W1, row(ee_b1), ee_W2, row(ee_b2),
        gin0_W1, row(gin0_b1), gin0_W2, row(gin0_b2),
        gin1_W1, row(gin1_b1), gin1_W2, row(gin1_b2),
        row(ln0_g), row(ln0_b), row(ln1_g), row(ln1_b),
        res1.reshape(1, 1), m_taps, row(b_eff),
        sf_W2, row(sf_b2), interpret=interpret)
    return final, mask


# speedup vs baseline: 1.1487x; 1.0017x over previous
import functools

import jax
import jax.numpy as jnp
from jax.experimental import pallas as pl
from jax.experimental.pallas import tpu as pltpu

B, S, E = 2, 256, 256
TI = 64
NTAP = 8


def _dot(a, b):
    return jax.lax.dot_general(a, b, (((1,), (0,)), ((), ())),
                               preferred_element_type=jnp.float32)


def _layer_norm(x, g, b, eps=1e-5):
    m = jnp.mean(x, axis=-1, keepdims=True)
    xc = x - m
    v = jnp.mean(xc * xc, axis=-1, keepdims=True)
    return xc * jax.lax.rsqrt(v + eps) * g + b


def _fused_kernel(adj_ref, x_ref, ee_W1_ref, ee_b1_ref, ee_W2_ref, ee_b2_ref,
                  gin0_W1_ref, gin0_b1_ref, gin0_W2_ref, gin0_b2_ref,
                  gin1_W1_ref, gin1_b1_ref, gin1_W2_ref, gin1_b2_ref,
                  ln0_g_ref, ln0_b_ref, ln1_g_ref, ln1_b_ref, res1_ref,
                  m_ref, beff_ref, sf_W2_ref, sf_b2_ref,
                  final_ref, msum_ref, e_ref):
    x = x_ref[0]
    adj = adj_ref[0]

    a_rows = _dot(x, ee_W1_ref[:E, :]) + ee_b1_ref[...]
    b_rows = _dot(x, ee_W1_ref[E:, :])
    w2 = ee_W2_ref[...]
    b2 = ee_b2_ref[...]
    for i0 in range(0, S, TI):
        h = jax.nn.relu(a_rows[i0:i0 + TI, None, :] + b_rows[None, :, :])
        h = h.reshape(TI * S, E)
        t = jnp.tanh(_dot(h, w2) + b2)
        e_ref[i0:i0 + TI, :] = jnp.mean(t, axis=-1).reshape(TI, S)
    e = e_ref[...]
    adj_e = adj * (1.0 + e)

    msum_ref[0] = (jnp.sum(adj_e, axis=1, keepdims=True)
                   + jnp.sum(adj_e, axis=0).reshape(S, 1)) == 0.0

    zpad = jnp.zeros((4, E), jnp.float32)
    xp = jnp.concatenate([zpad, x, zpad], axis=0)
    pr = beff_ref[...]
    for j in range(NTAP):
        pr = pr + _dot(xp[j:j + S, :], m_ref[j])
    ms = _dot(jax.nn.relu(pr), sf_W2_ref[...]) + sf_b2_ref[...]

    gin_in0 = x + _dot(adj_e, x)
    lo = _dot(jax.nn.relu(_dot(gin_in0, gin0_W1_ref[...]) + gin0_b1_ref[...]),
              gin0_W2_ref[...]) + gin0_b2_ref[...]
    r0 = jax.nn.relu(_layer_norm(lo, ln0_g_ref[...], ln0_b_ref[...]))

    gin_in1 = r0 + _dot(adj_e, r0)
    lo = _dot(jax.nn.relu(_dot(gin_in1, gin1_W1_ref[...]) + gin1_b1_ref[...]),
              gin1_W2_ref[...]) + gin1_b2_ref[...]
    rw = jax.nn.sigmoid(res1_ref[0, 0])
    lo = rw * lo + (1.0 - rw) * r0
    out = jax.nn.relu(_layer_norm(lo, ln1_g_ref[...], ln1_b_ref[...]))

    final_ref[0] = jnp.concatenate([out, ms], axis=-1)


@functools.partial(jax.jit, static_argnames=("interpret",))
def _run(adj, inputs, ee_W1, ee_b1, ee_W2, ee_b2, gin0_W1, gin0_b1, gin0_W2,
         gin0_b2, gin1_W1, gin1_b1, gin1_W2, gin1_b2, ln0_g, ln0_b, ln1_g,
         ln1_b, res1, m_taps, b_eff, sf_W2, sf_b2, interpret=False):
    def full(shape):
        return pl.BlockSpec(shape, lambda b: (0,) * len(shape))

    in_specs = [
        pl.BlockSpec((1, S, S), lambda b: (b, 0, 0)),
        pl.BlockSpec((1, S, E), lambda b: (b, 0, 0)),
        full((2 * E, E)), full((1, E)), full((E, E)), full((1, E)),
        full((E, E)), full((1, E)), full((E, E)), full((1, E)),
        full((E, E)), full((1, E)), full((E, E)), full((1, E)),
        full((1, E)), full((1, E)), full((1, E)), full((1, E)),
        full((1, 1)),
        full((NTAP, E, E)), full((1, E)), full((E, E)), full((1, E)),
    ]
    out_specs = [
        pl.BlockSpec((1, S, 2 * E), lambda b: (b, 0, 0)),
        pl.BlockSpec((1, S, 1), lambda b: (b, 0, 0)),
    ]
    final, msum = pl.pallas_call(
        _fused_kernel,
        grid=(B,),
        in_specs=in_specs,
        out_specs=out_specs,
        out_shape=[
            jax.ShapeDtypeStruct((B, S, 2 * E), jnp.float32),
            jax.ShapeDtypeStruct((B, S, 1), jnp.bool_),
        ],
        scratch_shapes=[pltpu.VMEM((S, S), jnp.float32)],
        compiler_params=pltpu.CompilerParams(
            dimension_semantics=("arbitrary",)),
        interpret=interpret,
    )(adj, inputs, ee_W1, ee_b1, ee_W2, ee_b2, gin0_W1, gin0_b1, gin0_W2,
      gin0_b2, gin1_W1, gin1_b1, gin1_W2, gin1_b2, ln0_g, ln0_b, ln1_g,
      ln1_b, res1, m_taps, b_eff, sf_W2, sf_b2)
    return final, msum


def kernel(adj, inputs, ee_W1, ee_b1, ee_W2, ee_b2, gin0_W1, gin0_b1,
           gin0_W2, gin0_b2, gin1_W1, gin1_b1, gin1_W2, gin1_b2, ln0_g,
           ln0_b, ln1_g, ln1_b, res0, res1, conv1_W, conv1_b, conv2_W,
           conv2_b, conv3_W, conv3_b, sf_W1, sf_b1, sf_W2, sf_b2,
           interpret=False):
    row = lambda v: v.reshape(1, E)
    m_taps = jnp.einsum('oit,oe->tie', conv3_W, sf_W1[2 * E:])
    b_eff = sf_b1 + conv3_b @ sf_W1[2 * E:]
    for wk, bk, pk, off in ((conv1_W, conv1_b, 1, 0),
                            (conv2_W, conv2_b, 2, E)):
        k = wk.shape[2]
        sf = sf_W1[off:off + E]
        m_taps = m_taps.at[4 - pk:4 - pk + k].add(
            jnp.einsum('oit,oe->tie', wk, sf))
        b_eff = b_eff + bk @ sf
    final, mask = _run(
        adj, inputs, ee_W1, row(ee_b1), ee_W2, row(ee_b2),
        gin0_W1, row(gin0_b1), gin0_W2, row(gin0_b2),
        gin1_W1, row(gin1_b1), gin1_W2, row(gin1_b2),
        row(ln0_g), row(ln0_b), row(ln1_g), row(ln1_b),
        res1.reshape(1, 1), m_taps, row(b_eff),
        sf_W2, row(sf_b2), interpret=interpret)
    return final, mask
